# baseline (speedup 1.0000x reference)
import jax
import jax.numpy as jnp
from jax.experimental import pallas as pl
from jax.experimental.pallas import tpu as pltpu

_H, _W = 50, 38
_Q = _H * _W
_QP = 1920
_C = 512
_NL, _NS = 36, 18
_MARG = 64
_SFW = 2048


def _body(x2_ref, w2_ref, lw_ref, sw_ref, b1_ref, lb_ref, sb_ref,
          loc_ref, score_ref, sf_ref, cws_ref):
    sf_ref[:, :_MARG] = jnp.zeros((_C, _MARG), jnp.bfloat16)
    sf_ref[:, _MARG + _Q:] = jnp.zeros((_C, _SFW - _MARG - _Q), jnp.bfloat16)
    sf_ref[:, _MARG:_MARG + _Q] = x2_ref[:].astype(jnp.bfloat16)

    cww = jnp.concatenate(
        [jnp.dot(lw_ref[:], w2_ref[:], preferred_element_type=jnp.float32),
         jnp.dot(sw_ref[:], w2_ref[:], preferred_element_type=jnp.float32)],
        axis=0)
    cws_ref[:, :_NL + _NS] = cww.T
    bias = jnp.concatenate(
        [jnp.dot(lw_ref[:], b1_ref[:],
                 preferred_element_type=jnp.float32) + lb_ref[:],
         jnp.dot(sw_ref[:], b1_ref[:],
                 preferred_element_type=jnp.float32) + sb_ref[:]],
        axis=0)

    q = jax.lax.broadcasted_iota(jnp.int32, (1, _QP), 1)
    wcol = q - (q // _W) * _W
    mask_l = (wcol > 0).astype(jnp.float32)
    mask_r = (wcol < _W - 1).astype(jnp.float32)

    acc = jnp.zeros((_NL + _NS, _QP), jnp.float32)
    for ky in range(3):
        for kx in range(3):
            t = ky * 3 + kx
            delta = (ky - 1) * _W + (kx - 1)
            wt = cws_ref[pl.ds(t, _C, 9), :][:, :_NL + _NS].T
            ft = sf_ref[:, _MARG + delta:_MARG + delta + _QP]
            contr = jnp.dot(wt.astype(jnp.bfloat16), ft,
                            preferred_element_type=jnp.float32)
            if kx == 0:
                contr = contr * mask_l
            elif kx == 2:
                contr = contr * mask_r
            acc = acc + contr
    acc = acc + bias
    loc_ref[:] = acc[:_NL, :_Q]
    score_ref[:] = acc[_NL:, :_Q]


def kernel(out_map, conv1_w, conv1_b, loc_w, loc_b, score_w, score_b):
    x2 = out_map.reshape(_C, _Q)
    w2 = conv1_w.reshape(_C, _C * 9)
    lw = loc_w.reshape(_NL, _C)
    sw = score_w.reshape(_NS, _C)
    b1 = conv1_b.reshape(_C, 1)
    lb = loc_b.reshape(_NL, 1)
    sb = score_b.reshape(_NS, 1)

    loc, score = pl.pallas_call(
        _body,
        out_shape=(jax.ShapeDtypeStruct((_NL, _Q), jnp.float32),
                   jax.ShapeDtypeStruct((_NS, _Q), jnp.float32)),
        scratch_shapes=[
            pltpu.VMEM((_C, _SFW), jnp.bfloat16),
            pltpu.VMEM((_C * 9, 128), jnp.float32),
        ],
    )(x2, w2, lw, sw, b1, lb, sb)

    return (loc.reshape(1, _NL, _H, _W), score.reshape(1, _NS, _H, _W))


# speedup vs baseline: 1.3029x; 1.3029x over previous
import jax
import jax.numpy as jnp
from jax.experimental import pallas as pl
from jax.experimental.pallas import tpu as pltpu

_H, _W = 50, 38
_Q = _H * _W
_QP = 1920
_C = 512
_NL, _NS = 36, 18
_MARG = 64
_SFW = 2048


def _body(x2_ref, w2_ref, lw_ref, sw_ref, b1_ref, lb_ref, sb_ref,
          loc_ref, score_ref, sf_ref, cws_ref):
    sf_ref[:, :_MARG] = jnp.zeros((_C, _MARG), jnp.bfloat16)
    sf_ref[:, _MARG + _Q:] = jnp.zeros((_C, _SFW - _MARG - _Q), jnp.bfloat16)
    sf_ref[:, _MARG:_MARG + _Q] = x2_ref[:]

    lwf = lw_ref[:]
    swf = sw_ref[:]
    cww = jnp.concatenate(
        [jnp.dot(lwf.astype(jnp.bfloat16), w2_ref[:],
                 preferred_element_type=jnp.float32),
         jnp.dot(swf.astype(jnp.bfloat16), w2_ref[:],
                 preferred_element_type=jnp.float32)],
        axis=0)
    cws_ref[:, :_NL + _NS] = cww.T

    b1 = b1_ref[:]
    bias = jnp.concatenate(
        [jnp.sum(lwf * b1, axis=1, keepdims=True)
         + jnp.reshape(lb_ref[:], (_NL, 1)),
         jnp.sum(swf * b1, axis=1, keepdims=True)
         + jnp.reshape(sb_ref[:], (_NS, 1))],
        axis=0)

    q = jax.lax.broadcasted_iota(jnp.int32, (1, _QP), 1)
    wcol = q - (q // _W) * _W
    mask_l = (wcol > 0).astype(jnp.float32)
    mask_r = (wcol < _W - 1).astype(jnp.float32)

    acc = jnp.zeros((_NL + _NS, _QP), jnp.float32)
    for ky in range(3):
        for kx in range(3):
            t = ky * 3 + kx
            delta = (ky - 1) * _W + (kx - 1)
            wt = cws_ref[pl.ds(t, _C, 9), :][:, :_NL + _NS].T
            ft = sf_ref[:, _MARG + delta:_MARG + delta + _QP]
            contr = jnp.dot(wt.astype(jnp.bfloat16), ft,
                            preferred_element_type=jnp.float32)
            if kx == 0:
                contr = contr * mask_l
            elif kx == 2:
                contr = contr * mask_r
            acc = acc + contr
    acc = acc + bias
    loc_ref[:] = acc[:_NL, :_Q]
    score_ref[:] = acc[_NL:, :_Q]


def kernel(out_map, conv1_w, conv1_b, loc_w, loc_b, score_w, score_b):
    x2 = out_map.reshape(_C, _Q).astype(jnp.bfloat16)
    w2 = conv1_w.reshape(_C, _C * 9).astype(jnp.bfloat16)
    lw = loc_w.reshape(_NL, _C)
    sw = score_w.reshape(_NS, _C)
    b1 = conv1_b.reshape(1, _C)

    loc, score = pl.pallas_call(
        _body,
        out_shape=(jax.ShapeDtypeStruct((_NL, _Q), jnp.float32),
                   jax.ShapeDtypeStruct((_NS, _Q), jnp.float32)),
        scratch_shapes=[
            pltpu.VMEM((_C, _SFW), jnp.bfloat16),
            pltpu.VMEM((_C * 9, 128), jnp.float32),
        ],
    )(x2, w2, lw, sw, b1, loc_b, score_b)

    return (loc.reshape(1, _NL, _H, _W), score.reshape(1, _NS, _H, _W))


# speedup vs baseline: 2.1688x; 1.6646x over previous
import jax
import jax.numpy as jnp
from jax.experimental import pallas as pl
from jax.experimental.pallas import tpu as pltpu

_H, _W = 50, 38
_Q = _H * _W
_QP = 1920
_C = 512
_NL, _NS = 36, 18
_NH = _NL + _NS
_MARG = 64
_SFH = 2048


def _body(x_ref, w_ref, lw_ref, sw_ref, b1_ref, lb_ref, sb_ref,
          locT_ref, scoT_ref, sf_ref):
    sf_ref[:_MARG, :] = jnp.zeros((_MARG, _C), jnp.bfloat16)
    sf_ref[_MARG + _Q:, :] = jnp.zeros((_SFH - _MARG - _Q, _C), jnp.bfloat16)
    sf_ref[_MARG:_MARG + _Q, :] = x_ref[:]

    h = jnp.concatenate([lw_ref[:], sw_ref[:]], axis=0)
    hb = jnp.concatenate([lb_ref[:], sb_ref[:]], axis=1)
    bias = jnp.sum(h.astype(jnp.float32) * b1_ref[:],
                   axis=1, keepdims=True).T + hb

    q = jax.lax.broadcasted_iota(jnp.int32, (_QP, 1), 0)
    wcol = q - (q // _W) * _W
    mask_l = (wcol > 0).astype(jnp.float32)
    mask_r = (wcol < _W - 1).astype(jnp.float32)

    sfb = sf_ref[:, :]
    acc = jnp.zeros((_QP, _NH), jnp.float32)
    for ky in range(3):
        for kx in range(3):
            t = ky * 3 + kx
            delta = (ky - 1) * _W + (kx - 1)
            cwwT = jnp.dot(h, w_ref[t],
                           preferred_element_type=jnp.float32).T
            p = jnp.dot(sfb, cwwT.astype(jnp.bfloat16),
                        preferred_element_type=jnp.float32)
            contr = p[_MARG + delta:_MARG + delta + _QP, :]
            if kx == 0:
                contr = contr * mask_l
            elif kx == 2:
                contr = contr * mask_r
            acc = acc + contr
    acc = acc + bias
    locT_ref[:] = acc[:_Q, :_NL]
    scoT_ref[:] = acc[:_Q, _NL:]


def kernel(out_map, conv1_w, conv1_b, loc_w, loc_b, score_w, score_b):
    xT = out_map.transpose(2, 3, 0, 1).reshape(_Q, _C).astype(jnp.bfloat16)
    w9 = conv1_w.transpose(2, 3, 0, 1).reshape(9, _C, _C)
    lw = loc_w.transpose(0, 2, 3, 1).reshape(_NL, _C).astype(jnp.bfloat16)
    sw = score_w.transpose(0, 2, 3, 1).reshape(_NS, _C).astype(jnp.bfloat16)
    b1 = conv1_b.reshape(1, _C)
    lb = loc_b.reshape(1, _NL)
    sb = score_b.reshape(1, _NS)

    locT, scoT = pl.pallas_call(
        _body,
        out_shape=(jax.ShapeDtypeStruct((_Q, _NL), jnp.float32),
                   jax.ShapeDtypeStruct((_Q, _NS), jnp.float32)),
        scratch_shapes=[
            pltpu.VMEM((_SFH, _C), jnp.bfloat16),
        ],
    )(xT, w9, lw, sw, b1, lb, sb)

    loc = locT.reshape(_H, _W, _NL).transpose(2, 0, 1)[None]
    score = scoT.reshape(_H, _W, _NS).transpose(2, 0, 1)[None]
    return (loc, score)


# speedup vs baseline: 4.0067x; 1.8474x over previous
import jax
import jax.numpy as jnp
from jax.experimental import pallas as pl
from jax.experimental.pallas import tpu as pltpu

_H, _W = 50, 38
_Q = _H * _W
_QP = 1920
_C = 512
_NL, _NS = 36, 18
_NH = _NL + _NS
_MARG = 64
_SFH = 2048


def _body(x_ref, w_ref, lw_ref, sw_ref, b1_ref, lb_ref, sb_ref,
          locT_ref, scoT_ref, sf_ref):
    sf_ref[:_MARG, :] = jnp.zeros((_MARG, _C), jnp.bfloat16)
    sf_ref[_MARG + _Q:, :] = jnp.zeros((_SFH - _MARG - _Q, _C), jnp.bfloat16)
    for g in range(4):
        colv = x_ref[pl.ds(g, _Q, 4), :]
        sf_ref[_MARG:_MARG + _Q, 128 * g:128 * (g + 1)] = (
            colv.astype(jnp.bfloat16))

    h = jnp.concatenate([lw_ref[:], sw_ref[:]], axis=0)
    hb = jnp.concatenate([lb_ref[:], sb_ref[:]], axis=1)
    bias = jnp.sum(h.astype(jnp.float32) * b1_ref[:],
                   axis=1, keepdims=True).T + hb

    q = jax.lax.broadcasted_iota(jnp.int32, (_QP, 1), 0)
    wcol = q - (q // _W) * _W
    mask_l = (wcol > 0).astype(jnp.float32)
    mask_r = (wcol < _W - 1).astype(jnp.float32)

    sfb = sf_ref[:, :]
    acc = jnp.zeros((_QP, _NH), jnp.float32)
    for ky in range(3):
        for kx in range(3):
            t = ky * 3 + kx
            delta = (ky - 1) * _W + (kx - 1)
            cwwT = jnp.dot(h, w_ref[t],
                           preferred_element_type=jnp.float32).T
            p = jnp.dot(sfb, cwwT.astype(jnp.bfloat16),
                        preferred_element_type=jnp.float32)
            contr = p[_MARG + delta:_MARG + delta + _QP, :]
            if kx == 0:
                contr = contr * mask_l
            elif kx == 2:
                contr = contr * mask_r
            acc = acc + contr
    acc = acc + bias
    locT_ref[:] = acc[:_Q, :_NL]
    scoT_ref[:] = acc[:_Q, _NL:]


def kernel(out_map, conv1_w, conv1_b, loc_w, loc_b, score_w, score_b):
    xT = out_map.transpose(2, 3, 0, 1).reshape(_Q * 4, _C // 4)
    w9 = conv1_w.transpose(2, 3, 0, 1).reshape(9, _C, _C)
    lw = loc_w.transpose(0, 2, 3, 1).reshape(_NL, _C).astype(jnp.bfloat16)
    sw = score_w.transpose(0, 2, 3, 1).reshape(_NS, _C).astype(jnp.bfloat16)
    b1 = conv1_b.reshape(1, _C)
    lb = loc_b.reshape(1, _NL)
    sb = score_b.reshape(1, _NS)

    locT, scoT = pl.pallas_call(
        _body,
        out_shape=(jax.ShapeDtypeStruct((_Q, _NL), jnp.float32),
                   jax.ShapeDtypeStruct((_Q, _NS), jnp.float32)),
        scratch_shapes=[
            pltpu.VMEM((_SFH, _C), jnp.bfloat16),
        ],
    )(xT, w9, lw, sw, b1, lb, sb)

    loc = locT.reshape(_H, _W, _NL).transpose(2, 0, 1)[None]
    score = scoT.reshape(_H, _W, _NS).transpose(2, 0, 1)[None]
    return (loc, score)
